# baseline (speedup 1.0000x reference)
import functools

import jax
import jax.numpy as jnp
from jax import lax
from jax.experimental import pallas as pl
from jax.experimental.pallas import tpu as pltpu
from jax.experimental.pallas import tpu_sc as plsc

N = 16384
C = 1000
N_BINS = 15
BR = 512
NBLK = N // BR
NSUB = 16
LANES = 16
PER_TILE = N // NSUB
VREGS = PER_TILE // LANES


def _tc_body(x_ref, lab_ref, conf_ref, acc_ref):
    x = x_ref[...]
    m = jnp.max(x, axis=1)
    col = lax.broadcasted_iota(jnp.int32, x.shape, 1)
    first = jnp.min(jnp.where(x == m[:, None], col, C), axis=1)
    acc = (first == lab_ref[0, 0, :]).astype(jnp.float32)
    conf_ref[0, 0, :] = m
    acc_ref[0, 0, :] = acc


_tc_call = pl.pallas_call(
    _tc_body,
    grid=(NBLK,),
    in_specs=[
        pl.BlockSpec((BR, C), lambda i: (i, 0)),
        pl.BlockSpec((1, 1, BR), lambda i: (i, 0, 0)),
    ],
    out_specs=[
        pl.BlockSpec((1, 1, BR), lambda i: (i, 0, 0)),
        pl.BlockSpec((1, 1, BR), lambda i: (i, 0, 0)),
    ],
    out_shape=[
        jax.ShapeDtypeStruct((NBLK, 1, BR), jnp.float32),
        jax.ShapeDtypeStruct((NBLK, 1, BR), jnp.float32),
    ],
)


@functools.cache
def _get_sc_hist():
    return pl.kernel(
        _sc_hist_body,
        out_type=jax.ShapeDtypeStruct((LANES,), jnp.float32),
        mesh=plsc.VectorSubcoreMesh(core_axis_name="c", subcore_axis_name="s"),
        scratch_types=[
            pltpu.VMEM((VREGS, LANES), jnp.float32),
            pltpu.VMEM((VREGS, LANES), jnp.float32),
            pltpu.VMEM((LANES, LANES), jnp.float32),
            pltpu.VMEM_SHARED((NSUB, LANES), jnp.float32),
            pltpu.VMEM((NSUB, LANES), jnp.float32),
            pltpu.VMEM((LANES,), jnp.float32),
        ],
        compiler_params=pltpu.CompilerParams(needs_layout_passes=False),
    )


def _sc_hist_body(conf_hbm, acc_hbm, out_hbm, conf_v, acc_v, hist_v,
                  shared, sums_v, row_v):
    cid = lax.axis_index("c")
    sid = lax.axis_index("s")

    @pl.when(cid == 0)
    def _core0():
        pltpu.sync_copy(conf_hbm.at[sid], conf_v)
        pltpu.sync_copy(acc_hbm.at[sid], acc_v)
        zeros = jnp.zeros((LANES,), jnp.float32)
        for r in range(LANES):
            hist_v[r] = zeros
        lane = lax.iota(jnp.int32, LANES)

        def body(i, carry):
            c = conf_v[i]
            a = acc_v[i]
            t = c * jnp.float32(N_BINS)
            bi = t.astype(jnp.int32)
            is_int = (bi.astype(jnp.float32) == t).astype(jnp.int32)
            b = jnp.minimum(jnp.maximum(bi - is_int, 0), N_BINS - 1)
            plsc.addupdate_scatter(hist_v, [lane, b], c - a, mask=c > 0.0)
            return carry

        lax.fori_loop(0, VREGS, body, 0)

        tot = hist_v[0]
        for r in range(1, LANES):
            tot = tot + hist_v[r]
        row_v[...] = tot
        pltpu.sync_copy(row_v, shared.at[sid])
        plsc.subcore_barrier()

        @pl.when(sid == 0)
        def _tile0():
            pltpu.sync_copy(shared, sums_v)
            g = sums_v[0]
            for r in range(1, NSUB):
                g = g + sums_v[r]
            ece = jnp.sum(jnp.abs(g)) * jnp.float32(1.0 / N)
            row_v[...] = jnp.broadcast_to(ece, (LANES,))
            pltpu.sync_copy(row_v, out_hbm)


def kernel(softmaxes, labels):
    conf3, acc3 = _tc_call(softmaxes, labels.reshape(NBLK, 1, BR))
    conf = conf3.reshape(NSUB, VREGS, LANES)
    acc = acc3.reshape(NSUB, VREGS, LANES)
    ece16 = _get_sc_hist()(conf, acc)
    return ece16[:1]


# speedup vs baseline: 1.1537x; 1.1537x over previous
import functools

import jax
import jax.numpy as jnp
from jax import lax
from jax.experimental import pallas as pl
from jax.experimental.pallas import tpu as pltpu
from jax.experimental.pallas import tpu_sc as plsc

N = 16384
C = 1000
N_BINS = 15
BR = 1024
NBLK = N // BR
NSUB = 16
LANES = 16
SUBL = BR // 128


def _tc_body(x_ref, lab_ref, conf_ref, acc_ref):
    x = x_ref[...]
    m = jnp.max(x, axis=1)
    col = lax.broadcasted_iota(jnp.int32, x.shape, 1)
    first = jnp.min(jnp.where(x == m[:, None], col, C), axis=1)
    acc = (first.reshape(SUBL, 128) == lab_ref[...]).astype(jnp.float32)
    conf_ref[...] = m.reshape(SUBL, 128)
    acc_ref[...] = acc


_tc_call = pl.pallas_call(
    _tc_body,
    grid=(NBLK,),
    in_specs=[
        pl.BlockSpec((BR, C), lambda i: (i, 0)),
        pl.BlockSpec((SUBL, 128), lambda i: (i, 0)),
    ],
    out_specs=[
        pl.BlockSpec((SUBL, 128), lambda i: (i, 0)),
        pl.BlockSpec((SUBL, 128), lambda i: (i, 0)),
    ],
    out_shape=[
        jax.ShapeDtypeStruct((128, 128), jnp.float32),
        jax.ShapeDtypeStruct((128, 128), jnp.float32),
    ],
)


@functools.cache
def _get_sc_hist():
    return pl.kernel(
        _sc_hist_body,
        out_type=jax.ShapeDtypeStruct((LANES,), jnp.float32),
        mesh=plsc.VectorSubcoreMesh(core_axis_name="c", subcore_axis_name="s"),
        scratch_types=[
            pltpu.VMEM((SUBL, 128), jnp.float32),
            pltpu.VMEM((SUBL, 128), jnp.float32),
            pltpu.VMEM((LANES, LANES), jnp.float32),
            pltpu.VMEM_SHARED((NSUB, LANES), jnp.float32),
            pltpu.VMEM((NSUB, LANES), jnp.float32),
            pltpu.VMEM((LANES,), jnp.float32),
        ],
        compiler_params=pltpu.CompilerParams(needs_layout_passes=False),
    )


def _sc_hist_body(conf_hbm, acc_hbm, out_hbm, conf_v, acc_v, hist_v,
                  shared, sums_v, row_v):
    cid = lax.axis_index("c")
    sid = lax.axis_index("s")

    @pl.when(cid == 0)
    def _core0():
        pltpu.sync_copy(conf_hbm.at[pl.ds(sid * SUBL, SUBL)], conf_v)
        pltpu.sync_copy(acc_hbm.at[pl.ds(sid * SUBL, SUBL)], acc_v)
        zeros = jnp.zeros((LANES,), jnp.float32)
        for r in range(LANES):
            hist_v[r] = zeros
        lane = lax.iota(jnp.int32, LANES)

        for r in range(SUBL):
            for j in range(128 // LANES):
                c = conf_v[r, pl.ds(j * LANES, LANES)]
                a = acc_v[r, pl.ds(j * LANES, LANES)]
                t = c * jnp.float32(N_BINS)
                bi = t.astype(jnp.int32)
                is_int = (bi.astype(jnp.float32) == t).astype(jnp.int32)
                b = jnp.minimum(jnp.maximum(bi - is_int, 0), N_BINS - 1)
                plsc.addupdate_scatter(hist_v, [lane, b], c - a, mask=c > 0.0)

        tot = hist_v[0]
        for r in range(1, LANES):
            tot = tot + hist_v[r]
        row_v[...] = tot
        pltpu.sync_copy(row_v, shared.at[sid])
        plsc.subcore_barrier()

        @pl.when(sid == 0)
        def _tile0():
            pltpu.sync_copy(shared, sums_v)
            g = sums_v[0]
            for r in range(1, NSUB):
                g = g + sums_v[r]
            ece = jnp.sum(jnp.abs(g)) * jnp.float32(1.0 / N)
            row_v[...] = jnp.broadcast_to(ece, (LANES,))
            pltpu.sync_copy(row_v, out_hbm)


def kernel(softmaxes, labels):
    conf2, acc2 = _tc_call(softmaxes, labels.reshape(128, 128))
    ece16 = _get_sc_hist()(conf2, acc2)
    return ece16[:1]


# speedup vs baseline: 3.0599x; 2.6523x over previous
import functools

import jax
import jax.numpy as jnp
from jax import lax
from jax.experimental import pallas as pl
from jax.experimental.pallas import tpu as pltpu
from jax.experimental.pallas import tpu_sc as plsc

N = 16384
C = 1000
N_BINS = 15
BR = 2048
NBLK = N // BR
NSUB = 16
LANES = 16
SUBL = BR // 128
SCR = 128 // NSUB


def _tc_body(xt_ref, lab_ref, v_ref):
    x = xt_ref[...]
    m = jnp.max(x, axis=0)
    row = lax.broadcasted_iota(jnp.int32, x.shape, 0)
    first = jnp.min(jnp.where(x == m[None, :], row, C), axis=0)
    acc = (first.reshape(SUBL, 128) == lab_ref[...]).astype(jnp.float32)
    v_ref[...] = m.reshape(SUBL, 128) - acc


_tc_call = pl.pallas_call(
    _tc_body,
    grid=(NBLK,),
    in_specs=[
        pl.BlockSpec((C, BR), lambda i: (0, i)),
        pl.BlockSpec((SUBL, 128), lambda i: (i, 0)),
    ],
    out_specs=pl.BlockSpec((SUBL, 128), lambda i: (i, 0)),
    out_shape=jax.ShapeDtypeStruct((128, 128), jnp.float32),
)


@functools.cache
def _get_sc_hist():
    return pl.kernel(
        _sc_hist_body,
        out_type=jax.ShapeDtypeStruct((LANES,), jnp.float32),
        mesh=plsc.VectorSubcoreMesh(core_axis_name="c", subcore_axis_name="s",
                                    num_cores=1),
        scratch_types=[
            pltpu.VMEM((SCR, 128), jnp.float32),
            pltpu.VMEM((LANES, LANES), jnp.float32),
            pltpu.VMEM_SHARED((NSUB, LANES), jnp.float32),
            pltpu.VMEM((NSUB, LANES), jnp.float32),
            pltpu.VMEM((LANES,), jnp.float32),
        ],
        compiler_params=pltpu.CompilerParams(needs_layout_passes=False),
    )


def _sc_hist_body(v_hbm, out_hbm, v_v, hist_v, shared, sums_v, row_v):
    sid = lax.axis_index("s")

    if True:
        pltpu.sync_copy(v_hbm.at[pl.ds(sid * SCR, SCR)], v_v)
        zeros = jnp.zeros((LANES,), jnp.float32)
        for r in range(LANES):
            hist_v[r] = zeros
        lane = lax.iota(jnp.int32, LANES)

        def body(r, carry):
            for j in range(128 // LANES):
                v = v_v[r, pl.ds(j * LANES, LANES)]
                c = v + (v < 0.0).astype(jnp.float32)
                t = c * jnp.float32(N_BINS)
                bi = t.astype(jnp.int32)
                is_int = (bi.astype(jnp.float32) == t).astype(jnp.int32)
                b = jnp.minimum(jnp.maximum(bi - is_int, 0), N_BINS - 1)
                plsc.addupdate_scatter(hist_v, [lane, b], v, mask=c > 0.0)
            return carry

        lax.fori_loop(0, SCR, body, 0)

        tot = hist_v[0]
        for r in range(1, LANES):
            tot = tot + hist_v[r]
        row_v[...] = tot
        pltpu.sync_copy(row_v, shared.at[sid])
        plsc.subcore_barrier()

        @pl.when(sid == 0)
        def _tile0():
            pltpu.sync_copy(shared, sums_v)
            g = sums_v[0]
            for r in range(1, NSUB):
                g = g + sums_v[r]
            ece = jnp.sum(jnp.abs(g)) * jnp.float32(1.0 / N)
            row_v[...] = jnp.broadcast_to(ece, (LANES,))
            pltpu.sync_copy(row_v, out_hbm)


def kernel(softmaxes, labels):
    v2 = _tc_call(softmaxes.T, labels.reshape(128, 128))
    ece16 = _get_sc_hist()(v2)
    return ece16[:1]


# speedup vs baseline: 3.0680x; 1.0027x over previous
import functools

import jax
import jax.numpy as jnp
from jax import lax
from jax.experimental import pallas as pl
from jax.experimental.pallas import tpu as pltpu
from jax.experimental.pallas import tpu_sc as plsc

N = 16384
C = 1000
N_BINS = 15
BR = 2048
NBLK = N // BR
NSUB = 16
LANES = 16
SUBL = BR // 128
SCR = 128 // NSUB


def _tc_body(xt_ref, lab_ref, conf_ref, acc_ref):
    x = xt_ref[...]
    m = jnp.max(x, axis=0)
    row = lax.broadcasted_iota(jnp.int32, x.shape, 0)
    first = jnp.min(jnp.where(x == m[None, :], row, C), axis=0)
    acc = (first.reshape(SUBL, 128) == lab_ref[...]).astype(jnp.float32)
    conf_ref[...] = m.reshape(SUBL, 128)
    acc_ref[...] = acc


_tc_call = pl.pallas_call(
    _tc_body,
    grid=(NBLK,),
    in_specs=[
        pl.BlockSpec((C, BR), lambda i: (0, i)),
        pl.BlockSpec((SUBL, 128), lambda i: (i, 0)),
    ],
    out_specs=[
        pl.BlockSpec((SUBL, 128), lambda i: (i, 0)),
        pl.BlockSpec((SUBL, 128), lambda i: (i, 0)),
    ],
    out_shape=[
        jax.ShapeDtypeStruct((128, 128), jnp.float32),
        jax.ShapeDtypeStruct((128, 128), jnp.float32),
    ],
)


@functools.cache
def _get_sc_hist():
    return pl.kernel(
        _sc_hist_body,
        out_type=jax.ShapeDtypeStruct((LANES,), jnp.float32),
        mesh=plsc.VectorSubcoreMesh(core_axis_name="c", subcore_axis_name="s",
                                    num_cores=1),
        scratch_types=[
            pltpu.VMEM((SCR, 128), jnp.float32),
            pltpu.VMEM((SCR, 128), jnp.float32),
            pltpu.VMEM((LANES, LANES), jnp.float32),
            pltpu.VMEM_SHARED((NSUB, LANES), jnp.float32),
            pltpu.VMEM((NSUB, LANES), jnp.float32),
            pltpu.VMEM((LANES,), jnp.float32),
            pltpu.SemaphoreType.DMA,
            pltpu.SemaphoreType.DMA,
        ],
        compiler_params=pltpu.CompilerParams(needs_layout_passes=False),
    )


def _sc_hist_body(conf_hbm, acc_hbm, out_hbm, conf_v, acc_v, hist_v,
                  shared, sums_v, row_v, sem1, sem2):
    sid = lax.axis_index("s")

    if True:
        cp1 = pltpu.async_copy(conf_hbm.at[pl.ds(sid * SCR, SCR)], conf_v, sem1)
        cp2 = pltpu.async_copy(acc_hbm.at[pl.ds(sid * SCR, SCR)], acc_v, sem2)
        cp1.wait()
        cp2.wait()
        zeros = jnp.zeros((LANES,), jnp.float32)
        for r in range(LANES):
            hist_v[r] = zeros
        lane = lax.iota(jnp.int32, LANES)

        def body(r, carry):
            for j in range(128 // LANES):
                c = conf_v[r, pl.ds(j * LANES, LANES)]
                a = acc_v[r, pl.ds(j * LANES, LANES)]
                t = c * jnp.float32(N_BINS)
                bi = t.astype(jnp.int32)
                is_int = (bi.astype(jnp.float32) == t).astype(jnp.int32)
                b = jnp.minimum(jnp.maximum(bi - is_int, 0), N_BINS - 1)
                plsc.addupdate_scatter(hist_v, [lane, b], c - a, mask=c > 0.0)
            return carry

        lax.fori_loop(0, SCR, body, 0)

        tot = hist_v[0]
        for r in range(1, LANES):
            tot = tot + hist_v[r]
        row_v[...] = tot
        pltpu.sync_copy(row_v, shared.at[sid])
        plsc.subcore_barrier()

        @pl.when(sid == 0)
        def _tile0():
            pltpu.sync_copy(shared, sums_v)
            g = sums_v[0]
            for r in range(1, NSUB):
                g = g + sums_v[r]
            ece = jnp.sum(jnp.abs(g)) * jnp.float32(1.0 / N)
            row_v[...] = jnp.broadcast_to(ece, (LANES,))
            pltpu.sync_copy(row_v, out_hbm)


def kernel(softmaxes, labels):
    conf2, acc2 = _tc_call(softmaxes.T, labels.reshape(128, 128))
    ece16 = _get_sc_hist()(conf2, acc2)
    return ece16[:1]


# speedup vs baseline: 3.0713x; 1.0011x over previous
import functools

import jax
import jax.numpy as jnp
from jax import lax
from jax.experimental import pallas as pl
from jax.experimental.pallas import tpu as pltpu
from jax.experimental.pallas import tpu_sc as plsc

N = 16384
C = 1000
N_BINS = 15
BR = 2048
NBLK = N // BR
NSUB = 16
LANES = 16
SUBL = BR // 128
SCR = 128 // NSUB


def _tc_body(xt_ref, lab_ref, conf_ref, acc_ref):
    x = xt_ref[...]
    m = jnp.max(x, axis=0)
    row = lax.broadcasted_iota(jnp.int32, x.shape, 0)
    first = jnp.min(jnp.where(x == m[None, :], row, C), axis=0)
    acc = (first.reshape(SUBL, 128) == lab_ref[...]).astype(jnp.float32)
    conf_ref[...] = m.reshape(SUBL, 128)
    acc_ref[...] = acc


_tc_call = pl.pallas_call(
    _tc_body,
    grid=(NBLK,),
    in_specs=[
        pl.BlockSpec((C, BR), lambda i: (0, i)),
        pl.BlockSpec((SUBL, 128), lambda i: (i, 0)),
    ],
    out_specs=[
        pl.BlockSpec((SUBL, 128), lambda i: (i, 0)),
        pl.BlockSpec((SUBL, 128), lambda i: (i, 0)),
    ],
    out_shape=[
        jax.ShapeDtypeStruct((128, 128), jnp.float32),
        jax.ShapeDtypeStruct((128, 128), jnp.float32),
    ],
)


@functools.cache
def _get_sc_hist():
    return pl.kernel(
        _sc_hist_body,
        out_type=jax.ShapeDtypeStruct((LANES,), jnp.float32),
        mesh=plsc.VectorSubcoreMesh(core_axis_name="c", subcore_axis_name="s",
                                    num_cores=1),
        scratch_types=[
            pltpu.VMEM((SCR, 128), jnp.float32),
            pltpu.VMEM((SCR, 128), jnp.float32),
            pltpu.VMEM((LANES, LANES), jnp.float32),
            pltpu.VMEM_SHARED((NSUB, LANES), jnp.float32),
            pltpu.VMEM((NSUB, LANES), jnp.float32),
            pltpu.VMEM((LANES,), jnp.float32),
            pltpu.SemaphoreType.DMA,
            pltpu.SemaphoreType.DMA,
        ],
        compiler_params=pltpu.CompilerParams(needs_layout_passes=False),
    )


def _sc_hist_body(conf_hbm, acc_hbm, out_hbm, conf_v, acc_v, hist_v,
                  shared, sums_v, row_v, sem1, sem2):
    sid = lax.axis_index("s")
    cp1 = pltpu.async_copy(conf_hbm.at[pl.ds(sid * SCR, SCR)], conf_v, sem1)
    cp2 = pltpu.async_copy(acc_hbm.at[pl.ds(sid * SCR, SCR)], acc_v, sem2)
    cp1.wait()
    cp2.wait()
    zeros = jnp.zeros((LANES,), jnp.float32)
    for r in range(LANES):
        hist_v[r] = zeros
    lane = lax.iota(jnp.int32, LANES)

    def body(r, carry):
        for j in range(128 // LANES):
            c = conf_v[r, pl.ds(j * LANES, LANES)]
            a = acc_v[r, pl.ds(j * LANES, LANES)]
            t = c * jnp.float32(N_BINS)
            bi = t.astype(jnp.int32)
            is_int = (bi.astype(jnp.float32) == t).astype(jnp.int32)
            b = jnp.minimum(jnp.maximum(bi - is_int, 0), N_BINS - 1)
            plsc.addupdate_scatter(hist_v, [lane, b], c - a, mask=c > 0.0)
        return carry

    lax.fori_loop(0, SCR, body, 0)

    tot = hist_v[0]
    for r in range(1, LANES):
        tot = tot + hist_v[r]
    row_v[...] = tot
    pltpu.sync_copy(row_v, shared.at[sid])
    plsc.subcore_barrier()

    @pl.when(sid == 0)
    def _tile0():
        pltpu.sync_copy(shared, sums_v)
        g = sums_v[0]
        for r in range(1, NSUB):
            g = g + sums_v[r]
        ece = jnp.sum(jnp.abs(g)) * jnp.float32(1.0 / N)
        row_v[...] = jnp.broadcast_to(ece, (LANES,))
        pltpu.sync_copy(row_v, out_hbm)


def kernel(softmaxes, labels):
    conf2, acc2 = _tc_call(softmaxes.T, labels.reshape(128, 128))
    ece16 = _get_sc_hist()(conf2, acc2)
    return ece16[:1]
